# baseline (speedup 1.0000x reference)
import functools
import math

import jax
import jax.numpy as jnp
from jax import lax
from jax.experimental import pallas as pl
from jax.experimental.pallas import tpu as pltpu
from jax.experimental.pallas import tpu_sc as plsc

VOCAB = 1000000
EMBED_DIM = 64
BATCH = 4096
HIST = 200

_TOT = BATCH * HIST
_NC = 2
_NS = 16
_NW = _NC * _NS
_PER_W = _TOT // _NW
_CH = 128
_NCHUNK = _PER_W // _CH
_SCALE = math.sqrt(EMBED_DIM)
_L = 16


def _emb_body(idx_hbm, tab_hbm, out_hbm, idx_v, rows_v, sem):
    wid = lax.axis_index("s") * _NC + lax.axis_index("c")
    base = wid * _PER_W

    def chunk(j, carry):
        off = base + j * _CH
        pltpu.sync_copy(idx_hbm.at[pl.ds(off, _CH)], idx_v)
        pltpu.async_copy(tab_hbm.at[idx_v], rows_v, sem).wait()

        def scale_row(r, c2):
            for c in range(EMBED_DIM // _L):
                sl = pl.ds(c * _L, _L)
                rows_v[r, sl] = rows_v[r, sl] * _SCALE
            return c2

        lax.fori_loop(0, _CH, scale_row, 0)
        pltpu.sync_copy(rows_v, out_hbm.at[pl.ds(off, _CH)])
        return carry

    lax.fori_loop(0, _NCHUNK, chunk, 0)


_mesh = plsc.VectorSubcoreMesh(core_axis_name="c", subcore_axis_name="s")

_emb = functools.partial(
    pl.kernel,
    mesh=_mesh,
    out_type=jax.ShapeDtypeStruct((_TOT, EMBED_DIM), jnp.float32),
    scratch_types=[
        pltpu.VMEM((_CH,), jnp.int32),
        pltpu.VMEM((_CH, EMBED_DIM), jnp.float32),
        pltpu.SemaphoreType.DMA,
    ],
    compiler_params=pltpu.CompilerParams(use_tc_tiling_on_sc=False),
)(_emb_body)


def kernel(x, table):
    flat = x.reshape(-1).astype(jnp.int32)
    out = _emb(flat, table)
    return out.reshape(BATCH, HIST, EMBED_DIM)


# speedup vs baseline: 1.2286x; 1.2286x over previous
import functools
import math

import jax
import jax.numpy as jnp
from jax import lax
from jax.experimental import pallas as pl
from jax.experimental.pallas import tpu as pltpu
from jax.experimental.pallas import tpu_sc as plsc

VOCAB = 1000000
EMBED_DIM = 64
BATCH = 4096
HIST = 200

_TOT = BATCH * HIST
_NC = 2
_NS = 16
_NW = _NC * _NS
_PER_W = _TOT // _NW
_CH = 128
_NCHUNK = _PER_W // _CH
_SCALE = math.sqrt(EMBED_DIM)
_L = 16


def _emb_body(idx_hbm, tab_hbm, out_hbm,
              idx_all, rows0, rows1, gsem0, gsem1, osem0, osem1):
    wid = lax.axis_index("s") * _NC + lax.axis_index("c")
    base = wid * _PER_W

    pltpu.sync_copy(idx_hbm.at[pl.ds(wid * _NCHUNK, _NCHUNK)], idx_all)

    bufs = ((rows0, gsem0, osem0), (rows1, gsem1, osem1))

    def start_gather(c, rows, gsem):
        pltpu.async_copy(tab_hbm.at[idx_all.at[c]], rows, gsem)

    def scale(rows):
        def scale_row(r, carry):
            for c in range(EMBED_DIM // _L):
                sl = pl.ds(c * _L, _L)
                rows[r, sl] = rows[r, sl] * _SCALE
            return carry

        lax.fori_loop(0, _CH, scale_row, 0, unroll=2)

    start_gather(0, rows0, gsem0)

    def step(j2, carry):
        for b in range(2):
            rows_b, gsem_b, osem_b = bufs[b]
            rows_o, gsem_o, osem_o = bufs[1 - b]
            cur = j2 * 2 + b

            @pl.when((cur >= 1) & (cur + 1 < _NCHUNK))
            def _():
                pltpu.make_async_copy(
                    rows_o, out_hbm.at[pl.ds(base, _CH)], osem_o).wait()

            @pl.when(cur + 1 < _NCHUNK)
            def _():
                start_gather(cur + 1, rows_o, gsem_o)

            pltpu.make_async_copy(
                tab_hbm.at[idx_all.at[cur]], rows_b, gsem_b).wait()
            scale(rows_b)
            pltpu.async_copy(
                rows_b, out_hbm.at[pl.ds(base + cur * _CH, _CH)], osem_b)
        return carry

    lax.fori_loop(0, _NCHUNK // 2, step, 0)

    pltpu.make_async_copy(rows0, out_hbm.at[pl.ds(base, _CH)], osem0).wait()
    pltpu.make_async_copy(rows1, out_hbm.at[pl.ds(base, _CH)], osem1).wait()


_mesh = plsc.VectorSubcoreMesh(core_axis_name="c", subcore_axis_name="s")

_emb = functools.partial(
    pl.kernel,
    mesh=_mesh,
    out_type=jax.ShapeDtypeStruct((_TOT, EMBED_DIM), jnp.float32),
    scratch_types=[
        pltpu.VMEM((_NCHUNK, _CH), jnp.int32),
        pltpu.VMEM((_CH, EMBED_DIM), jnp.float32),
        pltpu.VMEM((_CH, EMBED_DIM), jnp.float32),
        pltpu.SemaphoreType.DMA,
        pltpu.SemaphoreType.DMA,
        pltpu.SemaphoreType.DMA,
        pltpu.SemaphoreType.DMA,
    ],
    compiler_params=pltpu.CompilerParams(use_tc_tiling_on_sc=False),
)(_emb_body)


def kernel(x, table):
    flat = x.reshape(-1).astype(jnp.int32).reshape(_NW * _NCHUNK, _CH)
    out = _emb(flat, table)
    return out.reshape(BATCH, HIST, EMBED_DIM)


# speedup vs baseline: 1.2574x; 1.0235x over previous
import functools
import math

import jax
import jax.numpy as jnp
from jax import lax
from jax.experimental import pallas as pl
from jax.experimental.pallas import tpu as pltpu
from jax.experimental.pallas import tpu_sc as plsc

VOCAB = 1000000
EMBED_DIM = 64
BATCH = 4096
HIST = 200

_TOT = BATCH * HIST
_NC = 2
_NS = 16
_NW = _NC * _NS
_ROWS_W = BATCH // _NW
_PER_W = _ROWS_W * HIST
_G0 = 128
_G1 = HIST - _G0
_SCALE = math.sqrt(EMBED_DIM)
_L = 16


def _emb_body(idx_hbm, tab_hbm, out_hbm,
              idx_all, rows0, rows1, gsem0, gsem1, osem0, osem1):
    wid = lax.axis_index("s") * _NC + lax.axis_index("c")
    row0 = wid * _ROWS_W

    pltpu.sync_copy(idx_hbm.at[pl.ds(row0 * HIST, _PER_W)], idx_all)

    bufs = ((rows0, gsem0, osem0), (rows1, gsem1, osem1))

    def start_gather(r, rows, gsem):
        off = r * HIST
        pltpu.async_copy(
            tab_hbm.at[idx_all.at[pl.ds(off, _G0)]],
            rows.at[pl.ds(0, _G0)], gsem)
        pltpu.async_copy(
            tab_hbm.at[idx_all.at[pl.ds(off + _G0, _G1)]],
            rows.at[pl.ds(_G0, _G1)], gsem)

    def wait_gather(r, rows, gsem):
        off = r * HIST
        pltpu.make_async_copy(
            tab_hbm.at[idx_all.at[pl.ds(off, _G0)]],
            rows.at[pl.ds(0, _G0)], gsem).wait()
        pltpu.make_async_copy(
            tab_hbm.at[idx_all.at[pl.ds(off + _G0, _G1)]],
            rows.at[pl.ds(_G0, _G1)], gsem).wait()

    def scale(rows):
        def scale_row(r, carry):
            for c in range(EMBED_DIM // _L):
                sl = pl.ds(c * _L, _L)
                rows[r, sl] = rows[r, sl] * _SCALE
            return carry

        lax.fori_loop(0, HIST, scale_row, 0, unroll=2)

    start_gather(0, rows0, gsem0)

    def step(j2, carry):
        for b in range(2):
            rows_b, gsem_b, osem_b = bufs[b]
            rows_o, gsem_o, osem_o = bufs[1 - b]
            cur = j2 * 2 + b

            @pl.when((cur >= 1) & (cur + 1 < _ROWS_W))
            def _():
                pltpu.make_async_copy(
                    rows_o, out_hbm.at[row0], osem_o).wait()

            @pl.when(cur + 1 < _ROWS_W)
            def _():
                start_gather(cur + 1, rows_o, gsem_o)

            wait_gather(cur, rows_b, gsem_b)
            scale(rows_b)
            pltpu.async_copy(rows_b, out_hbm.at[row0 + cur], osem_b)
        return carry

    lax.fori_loop(0, _ROWS_W // 2, step, 0)

    pltpu.make_async_copy(rows0, out_hbm.at[row0], osem0).wait()
    pltpu.make_async_copy(rows1, out_hbm.at[row0], osem1).wait()


_mesh = plsc.VectorSubcoreMesh(core_axis_name="c", subcore_axis_name="s")

_emb = functools.partial(
    pl.kernel,
    mesh=_mesh,
    out_type=jax.ShapeDtypeStruct((BATCH, HIST, EMBED_DIM), jnp.float32),
    scratch_types=[
        pltpu.VMEM((_PER_W,), jnp.int32),
        pltpu.VMEM((HIST, EMBED_DIM), jnp.float32),
        pltpu.VMEM((HIST, EMBED_DIM), jnp.float32),
        pltpu.SemaphoreType.DMA,
        pltpu.SemaphoreType.DMA,
        pltpu.SemaphoreType.DMA,
        pltpu.SemaphoreType.DMA,
    ],
    compiler_params=pltpu.CompilerParams(use_tc_tiling_on_sc=False),
)(_emb_body)


def kernel(x, table):
    flat = x.reshape(-1).astype(jnp.int32)
    return _emb(flat, table)
